# baseline (speedup 1.0000x reference)
import functools

import jax
import jax.numpy as jnp
from jax import lax
from jax.experimental import pallas as pl
from jax.experimental.pallas import tpu as pltpu
from jax.experimental.pallas import tpu_sc as plsc

B = 16384
NF = 26
ED = 64
NW = 32
BPW = B // NW
CHUNK_B = 4
CHUNK_L = CHUNK_B * NF
NCHUNK = BPW // CHUNK_B
LANES = 16
EC = ED // LANES

GRP_B = 16
GRP_L = GRP_B * NF
NSUB = GRP_L // CHUNK_L
NGRP = BPW // GRP_B


def _emb_bag_body(table_hbm, idx_hbm, par_hbm, w4e_hbm, out_hbm,
                  idx_v, par_v, rows_v, w4_v, out_v, acc_v, sem):
    nc = lax.axis_size("c")
    wid = lax.axis_index("s") * nc + lax.axis_index("c")
    base_l = wid * BPW * NF

    pltpu.sync_copy(w4e_hbm, w4_v)

    def group_body(ch, carry):
        off = base_l + ch * GRP_L
        pltpu.sync_copy(par_hbm.at[pl.ds(off, GRP_L)], par_v)
        for g in range(NSUB):
            pltpu.sync_copy(
                idx_hbm.at[pl.ds(off + g * CHUNK_L, CHUNK_L)], idx_v.at[g])
        copies = [
            pltpu.async_copy(
                table_hbm.at[idx_v.at[g]],
                rows_v.at[pl.ds(g * CHUNK_L, CHUNK_L), :], sem)
            for g in range(NSUB)
        ]
        for cp in copies:
            cp.wait()

        zero = jnp.zeros((LANES,), jnp.float32)
        lane = lax.iota(jnp.int32, LANES)
        accs = [zero] * GRP_B
        for f in range(NF):
            ws = [w4_v[f, pl.ds(c * LANES, LANES)] for c in range(EC)]
            pvec = plsc.load_gather(par_v, [lane * NF + f])
            for b in range(GRP_B):
                a = accs[b]
                j = b * NF + f
                half = pvec[b]
                for c in range(EC):
                    a = a + rows_v[j, pl.ds(half + c * LANES, LANES)] * ws[c]
                accs[b] = a
        for b in range(GRP_B):
            acc_v[b, :] = accs[b]
        vec = zero
        for j in range(LANES):
            col = plsc.load_gather(
                acc_v, [lane, jnp.full((LANES,), j, jnp.int32)])
            vec = vec + col
        out_v[pl.ds(ch * GRP_B, GRP_B)] = vec
        return carry

    lax.fori_loop(0, NGRP, group_body, 0)
    pltpu.sync_copy(out_v, out_hbm.at[pl.ds(wid * BPW, BPW)])


@functools.cache
def _emb_bag_fn():
    mesh = plsc.VectorSubcoreMesh(core_axis_name="c", subcore_axis_name="s")
    return pl.kernel(
        _emb_bag_body,
        out_type=jax.ShapeDtypeStruct((B,), jnp.float32),
        mesh=mesh,
        compiler_params=pltpu.CompilerParams(
            needs_layout_passes=False, use_tc_tiling_on_sc=True),
        scratch_types=[
            pltpu.VMEM((NSUB, CHUNK_L), jnp.int32),
            pltpu.VMEM((GRP_L,), jnp.int32),
            pltpu.VMEM((GRP_L, 2 * ED), jnp.float32),
            pltpu.VMEM((NF, ED), jnp.float32),
            pltpu.VMEM((BPW,), jnp.float32),
            pltpu.VMEM((LANES, LANES), jnp.float32),
            pltpu.SemaphoreType.DMA,
        ],
    )


MLP_BLK = 1024


def _mlp_body(x_ref, w1_ref, b1_ref, w2_ref, b2_ref, w3_ref, b3_ref,
              w4d_ref, b4_ref, out_ref):
    x = x_ref[...]
    h1 = jnp.maximum(
        jnp.dot(x, w1_ref[...], preferred_element_type=jnp.float32)
        + b1_ref[...], 0.0)
    h2 = jnp.maximum(
        jnp.dot(h1, w2_ref[...], preferred_element_type=jnp.float32)
        + b2_ref[...], 0.0)
    v = jnp.dot(w3_ref[...], w4d_ref[...],
                preferred_element_type=jnp.float32)
    c = (jnp.dot(b3_ref[...], w4d_ref[...],
                 preferred_element_type=jnp.float32)
         + b4_ref[...])
    out_ref[...] = (
        jnp.dot(h2, v, preferred_element_type=jnp.float32) + c)


def _mlp(dense_in, W1, b1, W2, b2, W3, b3, w4d, b4):
    full = lambda s: pl.BlockSpec(s, lambda i: (0, 0))
    return pl.pallas_call(
        _mlp_body,
        grid=(B // MLP_BLK,),
        in_specs=[
            pl.BlockSpec((MLP_BLK, 13), lambda i: (i, 0)),
            full((13, 256)), full((1, 256)),
            full((256, 256)), full((1, 256)),
            full((256, 256)), full((1, 256)),
            full((256, 1)), full((1, 1)),
        ],
        out_specs=pl.BlockSpec((MLP_BLK, 1), lambda i: (i, 0)),
        out_shape=jax.ShapeDtypeStruct((B, 1), jnp.float32),
    )(dense_in, W1, b1.reshape(1, 256), W2, b2.reshape(1, 256),
      W3, b3.reshape(1, 256), w4d, b4.reshape(1, 1))


def kernel(dense_in, sparse_idx, W1, b1, W2, b2, W3, b3, W4, b4, table):
    idx32 = sparse_idx.astype(jnp.int32)
    pair_flat = (idx32 >> 1).reshape(-1)
    par_flat = ((idx32 & 1) * ED).reshape(-1)
    table2 = table.reshape(-1, 2 * ED)
    w4e = W4[: NF * ED, 0].reshape(NF, ED)
    w4d = W4[NF * ED :, :]
    emb = _emb_bag_fn()(table2, pair_flat, par_flat, w4e)
    dense = _mlp(dense_in, W1, b1, W2, b2, W3, b3, w4d, b4)
    return dense + emb[:, None]


# speedup vs baseline: 3.4152x; 3.4152x over previous
import functools

import jax
import jax.numpy as jnp
from jax import lax
from jax.experimental import pallas as pl
from jax.experimental.pallas import tpu as pltpu
from jax.experimental.pallas import tpu_sc as plsc

B = 16384
NF = 26
ED = 64
NV = 1000000
PF = 32
NW = 32
BPW = B // NW
LANES = 16

GRP_B = 16
GRP_L = GRP_B * NF
CH = 104
NSUB = GRP_L // CH
NGRP = BPW // GRP_B


def _bag_body(pflat_hbm, fidx_hbm, out_hbm, idx_v, val_v, out_v, sem):
    nc = lax.axis_size("c")
    wid = lax.axis_index("s") * nc + lax.axis_index("c")
    base_l = wid * BPW * NF

    def group_body(ch, carry):
        off = base_l + ch * GRP_L
        for g in range(NSUB):
            pltpu.sync_copy(
                fidx_hbm.at[pl.ds(off + g * CH, CH)], idx_v.at[g])
        copies = [
            pltpu.async_copy(
                pflat_hbm.at[idx_v.at[g]],
                val_v.at[pl.ds(g * CH, CH)], sem)
            for g in range(NSUB)
        ]
        for cp in copies:
            cp.wait()

        lane = lax.iota(jnp.int32, LANES)
        acc = jnp.zeros((LANES,), jnp.float32)
        for k in range(NF):
            acc = acc + plsc.load_gather(val_v, [lane * NF + k])
        out_v[pl.ds(ch * GRP_B, GRP_B)] = acc
        return carry

    lax.fori_loop(0, NGRP, group_body, 0)
    pltpu.sync_copy(out_v, out_hbm.at[pl.ds(wid * BPW, BPW)])


@functools.cache
def _bag_fn():
    mesh = plsc.VectorSubcoreMesh(core_axis_name="c", subcore_axis_name="s")
    return pl.kernel(
        _bag_body,
        out_type=jax.ShapeDtypeStruct((B,), jnp.float32),
        mesh=mesh,
        compiler_params=pltpu.CompilerParams(
            needs_layout_passes=False, use_tc_tiling_on_sc=False),
        scratch_types=[
            pltpu.VMEM((NSUB, CH), jnp.int32),
            pltpu.VMEM((GRP_L,), jnp.float32),
            pltpu.VMEM((BPW,), jnp.float32),
            pltpu.SemaphoreType.DMA,
        ],
    )


RCHUNK = 8192
NRBLK = -(-NV // RCHUNK)
NVP = NRBLK * RCHUNK


def _proj_body(tT_ref, w4e_ref, out_ref):
    tT = tT_ref[...]
    w = w4e_ref[...]
    pjT = lax.dot_general(
        w, tT, (((1,), (0,)), ((), ())),
        preferred_element_type=jnp.float32)
    out_ref[...] = pjT.reshape(NF, RCHUNK // 128, 128)


def _proj(tT, w4e):
    return pl.pallas_call(
        _proj_body,
        grid=(NRBLK,),
        in_specs=[
            pl.BlockSpec((ED, RCHUNK), lambda i: (0, i)),
            pl.BlockSpec((NF, ED), lambda i: (0, 0)),
        ],
        out_specs=pl.BlockSpec(
            (NF, RCHUNK // 128, 128), lambda i: (0, i, 0)),
        out_shape=jax.ShapeDtypeStruct(
            (NF, NVP // 128, 128), jnp.float32),
    )(tT, w4e)


MLP_BLK = 1024


def _mlp_body(x_ref, w1_ref, b1_ref, w2_ref, b2_ref, w3_ref, b3_ref,
              w4d_ref, b4_ref, out_ref):
    x = x_ref[...]
    h1 = jnp.maximum(
        jnp.dot(x, w1_ref[...], preferred_element_type=jnp.float32)
        + b1_ref[...], 0.0)
    h2 = jnp.maximum(
        jnp.dot(h1, w2_ref[...], preferred_element_type=jnp.float32)
        + b2_ref[...], 0.0)
    v = jnp.dot(w3_ref[...], w4d_ref[...],
                preferred_element_type=jnp.float32)
    c = (jnp.dot(b3_ref[...], w4d_ref[...],
                 preferred_element_type=jnp.float32)
         + b4_ref[...])
    out_ref[...] = (
        jnp.dot(h2, v, preferred_element_type=jnp.float32) + c)


def _mlp(dense_in, W1, b1, W2, b2, W3, b3, w4d, b4):
    full = lambda s: pl.BlockSpec(s, lambda i: (0, 0))
    return pl.pallas_call(
        _mlp_body,
        grid=(B // MLP_BLK,),
        in_specs=[
            pl.BlockSpec((MLP_BLK, 13), lambda i: (i, 0)),
            full((13, 256)), full((1, 256)),
            full((256, 256)), full((1, 256)),
            full((256, 256)), full((1, 256)),
            full((256, 1)), full((1, 1)),
        ],
        out_specs=pl.BlockSpec((MLP_BLK, 1), lambda i: (i, 0)),
        out_shape=jax.ShapeDtypeStruct((B, 1), jnp.float32),
    )(dense_in, W1, b1.reshape(1, 256), W2, b2.reshape(1, 256),
      W3, b3.reshape(1, 256), w4d, b4.reshape(1, 1))


def kernel(dense_in, sparse_idx, W1, b1, W2, b2, W3, b3, W4, b4, table):
    idx32 = sparse_idx.astype(jnp.int32)
    fidx = (idx32
            + jnp.arange(NF, dtype=jnp.int32)[None, :] * NVP).reshape(-1)
    tT = jnp.swapaxes(table, 0, 1)
    w4e = W4[: NF * ED, 0].reshape(NF, ED)
    w4d = W4[NF * ED :, :]
    proj2d = _proj(tT, w4e)
    pflat = proj2d.reshape(-1)
    emb = _bag_fn()(pflat, fidx)
    dense = _mlp(dense_in, W1, b1, W2, b2, W3, b3, w4d, b4)
    return dense + emb[:, None]


# speedup vs baseline: 4.3902x; 1.2855x over previous
import functools

import jax
import jax.numpy as jnp
from jax import lax
from jax.experimental import pallas as pl
from jax.experimental.pallas import tpu as pltpu
from jax.experimental.pallas import tpu_sc as plsc

B = 16384
NF = 26
ED = 64
NV = 1000000
PF = 32
NW = 32
BPW = B // NW
LANES = 16

GRP_B = 16
GRP_L = GRP_B * NF
CH = 104
NSUB = GRP_L // CH
NGRP = BPW // GRP_B


LPW = BPW * NF


def _bag_body(pflat_hbm, fidx_hbm, out_hbm, idx_v, val_v, out_v,
              sem0, sem1):
    nc = lax.axis_size("c")
    wid = lax.axis_index("s") * nc + lax.axis_index("c")
    base_l = wid * LPW

    pltpu.sync_copy(fidx_hbm.at[pl.ds(base_l, LPW)], idx_v)

    sems = (sem0, sem1)
    lane = lax.iota(jnp.int32, LANES)

    def fire(ch, slot):
        for g in range(NSUB):
            pltpu.async_copy(
                pflat_hbm.at[idx_v.at[pl.ds(ch * GRP_L + g * CH, CH)]],
                val_v.at[slot, pl.ds(g * CH, CH)], sems[slot])

    def wait_slot(slot):
        pltpu.make_async_copy(
            pflat_hbm.at[pl.ds(0, GRP_L)], val_v.at[slot], sems[slot]
        ).wait()

    def consume(ch, slot):
        srow = jnp.full((LANES,), slot, jnp.int32)
        acc = jnp.zeros((LANES,), jnp.float32)
        for k in range(NF):
            acc = acc + plsc.load_gather(val_v, [srow, lane * NF + k])
        out_v[pl.ds(ch * GRP_B, GRP_B)] = acc

    fire(0, 0)

    def body(i, carry):
        ch0 = 2 * i
        ch1 = ch0 + 1
        fire(ch1, 1)
        wait_slot(0)
        consume(ch0, 0)

        @pl.when(i < NGRP // 2 - 1)
        def _():
            fire(ch0 + 2, 0)

        wait_slot(1)
        consume(ch1, 1)
        return carry

    lax.fori_loop(0, NGRP // 2, body, 0)
    pltpu.sync_copy(out_v, out_hbm.at[pl.ds(wid * BPW, BPW)])


@functools.cache
def _bag_fn():
    mesh = plsc.VectorSubcoreMesh(core_axis_name="c", subcore_axis_name="s")
    return pl.kernel(
        _bag_body,
        out_type=jax.ShapeDtypeStruct((B,), jnp.float32),
        mesh=mesh,
        compiler_params=pltpu.CompilerParams(
            needs_layout_passes=False, use_tc_tiling_on_sc=False),
        scratch_types=[
            pltpu.VMEM((LPW,), jnp.int32),
            pltpu.VMEM((2, GRP_L), jnp.float32),
            pltpu.VMEM((BPW,), jnp.float32),
            pltpu.SemaphoreType.DMA,
            pltpu.SemaphoreType.DMA,
        ],
    )


RCHUNK = 8192
NRBLK = -(-NV // RCHUNK)
NVP = NRBLK * RCHUNK


def _proj_body(tT_ref, w4e_ref, out_ref):
    tT = tT_ref[...]
    w = w4e_ref[...]
    pjT = lax.dot_general(
        w, tT, (((1,), (0,)), ((), ())),
        preferred_element_type=jnp.float32)
    out_ref[...] = pjT.reshape(NF, RCHUNK // 128, 128)


def _proj(tT, w4e):
    return pl.pallas_call(
        _proj_body,
        grid=(NRBLK,),
        in_specs=[
            pl.BlockSpec((ED, RCHUNK), lambda i: (0, i)),
            pl.BlockSpec((NF, ED), lambda i: (0, 0)),
        ],
        out_specs=pl.BlockSpec(
            (NF, RCHUNK // 128, 128), lambda i: (0, i, 0)),
        out_shape=jax.ShapeDtypeStruct(
            (NF, NVP // 128, 128), jnp.float32),
    )(tT, w4e)


MLP_BLK = 1024


def _mlp_body(x_ref, w1_ref, b1_ref, w2_ref, b2_ref, w3_ref, b3_ref,
              w4d_ref, b4_ref, out_ref):
    x = x_ref[...]
    h1 = jnp.maximum(
        jnp.dot(x, w1_ref[...], preferred_element_type=jnp.float32)
        + b1_ref[...], 0.0)
    h2 = jnp.maximum(
        jnp.dot(h1, w2_ref[...], preferred_element_type=jnp.float32)
        + b2_ref[...], 0.0)
    v = jnp.dot(w3_ref[...], w4d_ref[...],
                preferred_element_type=jnp.float32)
    c = (jnp.dot(b3_ref[...], w4d_ref[...],
                 preferred_element_type=jnp.float32)
         + b4_ref[...])
    out_ref[...] = (
        jnp.dot(h2, v, preferred_element_type=jnp.float32) + c)


def _mlp(dense_in, W1, b1, W2, b2, W3, b3, w4d, b4):
    full = lambda s: pl.BlockSpec(s, lambda i: (0, 0))
    return pl.pallas_call(
        _mlp_body,
        grid=(B // MLP_BLK,),
        in_specs=[
            pl.BlockSpec((MLP_BLK, 13), lambda i: (i, 0)),
            full((13, 256)), full((1, 256)),
            full((256, 256)), full((1, 256)),
            full((256, 256)), full((1, 256)),
            full((256, 1)), full((1, 1)),
        ],
        out_specs=pl.BlockSpec((MLP_BLK, 1), lambda i: (i, 0)),
        out_shape=jax.ShapeDtypeStruct((B, 1), jnp.float32),
    )(dense_in, W1, b1.reshape(1, 256), W2, b2.reshape(1, 256),
      W3, b3.reshape(1, 256), w4d, b4.reshape(1, 1))


def kernel(dense_in, sparse_idx, W1, b1, W2, b2, W3, b3, W4, b4, table):
    idx32 = sparse_idx.astype(jnp.int32)
    fidx = (idx32
            + jnp.arange(NF, dtype=jnp.int32)[None, :] * NVP).reshape(-1)
    tT = jnp.swapaxes(table, 0, 1)
    w4e = W4[: NF * ED, 0].reshape(NF, ED)
    w4d = W4[NF * ED :, :]
    proj2d = _proj(tT, w4e)
    pflat = proj2d.reshape(-1)
    emb = _bag_fn()(pflat, fidx)
    dense = _mlp(dense_in, W1, b1, W2, b2, W3, b3, w4d, b4)
    return dense + emb[:, None]


# speedup vs baseline: 4.6292x; 1.0544x over previous
import functools

import jax
import jax.numpy as jnp
from jax import lax
from jax.experimental import pallas as pl
from jax.experimental.pallas import tpu as pltpu
from jax.experimental.pallas import tpu_sc as plsc

B = 16384
NF = 26
ED = 64
NV = 1000000
PF = 32
NW = 32
BPW = B // NW
LANES = 16

GRP_B = 16
GRP_L = GRP_B * NF
CH = 104
NSUB = GRP_L // CH
NGRP = BPW // GRP_B


QC = 128
NQ = BPW // QC


def _bag_body(pflat_hbm, fidx_hbm, out_hbm,
              idx_v, val_v, out_v, sem):
    nc = lax.axis_size("c")
    wid = lax.axis_index("s") * nc + lax.axis_index("c")
    base_b = wid * BPW

    pltpu.sync_copy(fidx_hbm.at[:, pl.ds(base_b, BPW)], idx_v)

    def fire(j, carry):
        f = j // NQ
        q = j % NQ
        pltpu.async_copy(
            pflat_hbm.at[idx_v.at[f, pl.ds(q * QC, QC)]],
            val_v.at[f, pl.ds(q * QC, QC)], sem)
        return carry

    lax.fori_loop(0, NF * NQ, fire, 0)

    def drain(j, carry):
        f = j // NQ
        q = j % NQ
        pltpu.make_async_copy(
            pflat_hbm.at[pl.ds(0, QC)],
            val_v.at[f, pl.ds(q * QC, QC)], sem).wait()
        return carry

    lax.fori_loop(0, NF * NQ, drain, 0)

    def reduce_body(t, carry):
        acc = val_v[0, pl.ds(t * LANES, LANES)]
        for f in range(1, NF):
            acc = acc + val_v[f, pl.ds(t * LANES, LANES)]
        out_v[pl.ds(t * LANES, LANES)] = acc
        return carry

    lax.fori_loop(0, BPW // LANES, reduce_body, 0)
    pltpu.sync_copy(out_v, out_hbm.at[pl.ds(base_b, BPW)])


@functools.cache
def _bag_fn():
    mesh = plsc.VectorSubcoreMesh(core_axis_name="c", subcore_axis_name="s")
    return pl.kernel(
        _bag_body,
        out_type=jax.ShapeDtypeStruct((B,), jnp.float32),
        mesh=mesh,
        compiler_params=pltpu.CompilerParams(
            needs_layout_passes=False, use_tc_tiling_on_sc=False),
        scratch_types=[
            pltpu.VMEM((NF, BPW), jnp.int32),
            pltpu.VMEM((NF, BPW), jnp.float32),
            pltpu.VMEM((BPW,), jnp.float32),
            pltpu.SemaphoreType.DMA,
        ],
    )


RCHUNK = 8192
NRBLK = -(-NV // RCHUNK)
NVP = NRBLK * RCHUNK


def _proj_body(tT_ref, w4e_ref, out_ref):
    tT = tT_ref[...]
    w = w4e_ref[...]
    pjT = lax.dot_general(
        w, tT, (((1,), (0,)), ((), ())),
        preferred_element_type=jnp.float32)
    out_ref[...] = pjT.reshape(NF, RCHUNK // 128, 128)


def _proj(tT, w4e):
    return pl.pallas_call(
        _proj_body,
        grid=(NRBLK,),
        in_specs=[
            pl.BlockSpec((ED, RCHUNK), lambda i: (0, i)),
            pl.BlockSpec((NF, ED), lambda i: (0, 0)),
        ],
        out_specs=pl.BlockSpec(
            (NF, RCHUNK // 128, 128), lambda i: (0, i, 0)),
        out_shape=jax.ShapeDtypeStruct(
            (NF, NVP // 128, 128), jnp.float32),
    )(tT, w4e)


MLP_BLK = 1024


def _mlp_body(x_ref, w1_ref, b1_ref, w2_ref, b2_ref, w3_ref, b3_ref,
              w4d_ref, b4_ref, out_ref):
    x = x_ref[...]
    h1 = jnp.maximum(
        jnp.dot(x, w1_ref[...], preferred_element_type=jnp.float32)
        + b1_ref[...], 0.0)
    h2 = jnp.maximum(
        jnp.dot(h1, w2_ref[...], preferred_element_type=jnp.float32)
        + b2_ref[...], 0.0)
    v = jnp.dot(w3_ref[...], w4d_ref[...],
                preferred_element_type=jnp.float32)
    c = (jnp.dot(b3_ref[...], w4d_ref[...],
                 preferred_element_type=jnp.float32)
         + b4_ref[...])
    out_ref[...] = (
        jnp.dot(h2, v, preferred_element_type=jnp.float32) + c)


def _mlp(dense_in, W1, b1, W2, b2, W3, b3, w4d, b4):
    full = lambda s: pl.BlockSpec(s, lambda i: (0, 0))
    return pl.pallas_call(
        _mlp_body,
        grid=(B // MLP_BLK,),
        in_specs=[
            pl.BlockSpec((MLP_BLK, 13), lambda i: (i, 0)),
            full((13, 256)), full((1, 256)),
            full((256, 256)), full((1, 256)),
            full((256, 256)), full((1, 256)),
            full((256, 1)), full((1, 1)),
        ],
        out_specs=pl.BlockSpec((MLP_BLK, 1), lambda i: (i, 0)),
        out_shape=jax.ShapeDtypeStruct((B, 1), jnp.float32),
    )(dense_in, W1, b1.reshape(1, 256), W2, b2.reshape(1, 256),
      W3, b3.reshape(1, 256), w4d, b4.reshape(1, 1))


def kernel(dense_in, sparse_idx, W1, b1, W2, b2, W3, b3, W4, b4, table):
    idx32 = sparse_idx.astype(jnp.int32)
    fidx2 = (jnp.swapaxes(idx32, 0, 1)
             + jnp.arange(NF, dtype=jnp.int32)[:, None] * NVP)
    tT = jnp.swapaxes(table, 0, 1)
    w4e = W4[: NF * ED, 0].reshape(NF, ED)
    w4d = W4[NF * ED :, :]
    proj2d = _proj(tT, w4e)
    pflat = proj2d.reshape(-1)
    emb = _bag_fn()(pflat, fidx2)
    dense = _mlp(dense_in, W1, b1, W2, b2, W3, b3, w4d, b4)
    return dense + emb[:, None]


# speedup vs baseline: 5.1309x; 1.1084x over previous
import functools

import jax
import jax.numpy as jnp
from jax import lax
from jax.experimental import pallas as pl
from jax.experimental.pallas import tpu as pltpu
from jax.experimental.pallas import tpu_sc as plsc

B = 16384
NF = 26
ED = 64
NV = 1000000
PF = 32
NW = 32
BPW = B // NW
LANES = 16

GRP_B = 16
GRP_L = GRP_B * NF
CH = 104
NSUB = GRP_L // CH
NGRP = BPW // GRP_B


QC = 128
NQ = BPW // QC


def _bag_body(pflat_hbm, fidx_hbm, out_hbm,
              idx_v, val_v, out_v, sem):
    nc = lax.axis_size("c")
    wid = lax.axis_index("s") * nc + lax.axis_index("c")
    base_b = wid * BPW

    pltpu.sync_copy(fidx_hbm.at[:, pl.ds(base_b, BPW)], idx_v)

    def fire(j, carry):
        f = j // NQ
        q = j % NQ
        pltpu.async_copy(
            pflat_hbm.at[idx_v.at[f, pl.ds(q * QC, QC)]],
            val_v.at[f, pl.ds(q * QC, QC)], sem)
        return carry

    lax.fori_loop(0, NF * NQ, fire, 0)

    def drain(j, carry):
        f = j // NQ
        q = j % NQ
        pltpu.make_async_copy(
            pflat_hbm.at[pl.ds(0, QC)],
            val_v.at[f, pl.ds(q * QC, QC)], sem).wait()
        return carry

    lax.fori_loop(0, NF * NQ, drain, 0)

    def reduce_body(t, carry):
        acc = val_v[0, pl.ds(t * LANES, LANES)]
        for f in range(1, NF):
            acc = acc + val_v[f, pl.ds(t * LANES, LANES)]
        out_v[pl.ds(t * LANES, LANES)] = acc
        return carry

    lax.fori_loop(0, BPW // LANES, reduce_body, 0)
    pltpu.sync_copy(out_v, out_hbm.at[pl.ds(base_b, BPW)])


@functools.cache
def _bag_fn():
    mesh = plsc.VectorSubcoreMesh(core_axis_name="c", subcore_axis_name="s")
    return pl.kernel(
        _bag_body,
        out_type=jax.ShapeDtypeStruct((B,), jnp.float32),
        mesh=mesh,
        compiler_params=pltpu.CompilerParams(
            needs_layout_passes=False, use_tc_tiling_on_sc=False),
        scratch_types=[
            pltpu.VMEM((NF, BPW), jnp.int32),
            pltpu.VMEM((NF, BPW), jnp.float32),
            pltpu.VMEM((BPW,), jnp.float32),
            pltpu.SemaphoreType.DMA,
        ],
    )


RCHUNK = 8192
NRBLK = -(-NV // RCHUNK)
NVP = NRBLK * RCHUNK


def _proj_body(tT_ref, w4e_ref, out_ref):
    tT = tT_ref[...]
    w = w4e_ref[...]
    pjT = lax.dot_general(
        w, tT, (((1,), (0,)), ((), ())),
        preferred_element_type=jnp.float32)
    out_ref[...] = pjT.reshape(NF, RCHUNK // 128, 128)


def _proj(tT, w4e):
    return pl.pallas_call(
        _proj_body,
        grid=(NRBLK,),
        in_specs=[
            pl.BlockSpec((ED, RCHUNK), lambda i: (0, i)),
            pl.BlockSpec((NF, ED), lambda i: (0, 0)),
        ],
        out_specs=pl.BlockSpec(
            (NF, RCHUNK // 128, 128), lambda i: (0, i, 0)),
        out_shape=jax.ShapeDtypeStruct(
            (NF, NVP // 128, 128), jnp.float32),
    )(tT, w4e)


MLP_BLK = 1024


def _mlp_body(x_ref, w1_ref, b1_ref, w2_ref, b2_ref, w3_ref, b3_ref,
              w4d_ref, b4_ref, out_ref):
    x = x_ref[...]
    cdot = lambda a, b: lax.dot_general(
        a, b, (((0,), (0,)), ((), ())),
        preferred_element_type=jnp.float32)
    h1 = jnp.maximum(
        lax.dot_general(w1_ref[...], x, (((0,), (1,)), ((), ())),
                        preferred_element_type=jnp.float32)
        + b1_ref[...], 0.0)
    h2 = jnp.maximum(cdot(w2_ref[...], h1) + b2_ref[...], 0.0)
    v = jnp.dot(w3_ref[...], w4d_ref[...],
                preferred_element_type=jnp.float32)
    c = (jnp.dot(b3_ref[...], w4d_ref[...],
                 preferred_element_type=jnp.float32)
         + b4_ref[...])
    out_ref[0] = cdot(v, h2) + c


def _mlp(dense_in, W1, b1, W2, b2, W3, b3, w4d, b4):
    full = lambda s: pl.BlockSpec(s, lambda i: (0, 0))
    return pl.pallas_call(
        _mlp_body,
        grid=(B // MLP_BLK,),
        in_specs=[
            pl.BlockSpec((MLP_BLK, 13), lambda i: (i, 0)),
            full((13, 256)), full((256, 1)),
            full((256, 256)), full((256, 1)),
            full((256, 256)), full((1, 256)),
            full((256, 1)), full((1, 1)),
        ],
        out_specs=pl.BlockSpec((1, 1, MLP_BLK), lambda i: (i, 0, 0)),
        out_shape=jax.ShapeDtypeStruct((B // MLP_BLK, 1, MLP_BLK),
                                       jnp.float32),
    )(dense_in, W1, b1.reshape(256, 1), W2, b2.reshape(256, 1),
      W3, b3.reshape(1, 256), w4d, b4.reshape(1, 1))


def kernel(dense_in, sparse_idx, W1, b1, W2, b2, W3, b3, W4, b4, table):
    idx32 = sparse_idx.astype(jnp.int32)
    fidx2 = (jnp.swapaxes(idx32, 0, 1)
             + jnp.arange(NF, dtype=jnp.int32)[:, None] * NVP)
    tT = jnp.swapaxes(table, 0, 1)
    w4e = W4[: NF * ED, 0].reshape(NF, ED)
    w4d = W4[NF * ED :, :]
    proj2d = _proj(tT, w4e)
    pflat = proj2d.reshape(-1)
    emb = _bag_fn()(pflat, fidx2)
    dense = _mlp(dense_in, W1, b1, W2, b2, W3, b3, w4d, b4)
    return (dense.reshape(B) + emb)[:, None]
